# baseline (speedup 1.0000x reference)
import jax
import jax.numpy as jnp
from jax.experimental import pallas as pl
from jax.experimental.pallas import tpu as pltpu

_B_ROWS = 32


def _nc_body(x_ref, ct_ref, o_ref, c2_ref, csq_ref):
    k = ct_ref.shape[1]
    length = x_ref.shape[2]

    @pl.when(pl.program_id(0) == 0)
    def _init():
        c2t = jnp.transpose(ct_ref[...] * -2.0)
        c2_ref[...] = c2t
        csq_ref[...] = jnp.sum(c2t * c2t, axis=1, keepdims=True) * 0.25

    c2 = c2_ref[...]
    csq = csq_ref[...]

    for r in range(_B_ROWS):
        xt = x_ref[r]
        mm2t = jax.lax.dot_general(
            c2, xt, (((1,), (0,)), ((), ())),
            preferred_element_type=jnp.float32)
        scores = mm2t + csq
        o_ref[r, :] = jnp.argmin(scores, axis=0).astype(jnp.float32)


def kernel(x, clusters):
    bs, length, dim = x.shape
    k = clusters.shape[0]
    xt = jnp.transpose(x, (0, 2, 1))
    ct = clusters.T
    grid = bs // _B_ROWS
    out = pl.pallas_call(
        _nc_body,
        grid=(grid,),
        in_specs=[
            pl.BlockSpec((_B_ROWS, dim, length), lambda i: (i, 0, 0)),
            pl.BlockSpec((dim, k), lambda i: (0, 0)),
        ],
        out_specs=pl.BlockSpec((_B_ROWS, length), lambda i: (i, 0)),
        out_shape=jax.ShapeDtypeStruct((bs, length), jnp.float32),
        scratch_shapes=[
            pltpu.VMEM((k, dim), jnp.float32),
            pltpu.VMEM((k, 1), jnp.float32),
        ],
    )(xt, ct)
    return out


# speedup vs baseline: 1.1110x; 1.1110x over previous
import jax
import jax.numpy as jnp
from jax.experimental import pallas as pl
from jax.experimental.pallas import tpu as pltpu

_B_ROWS = 16


def _nc_body(x_ref, ct_ref, o_ref, c2a_ref):
    length = x_ref.shape[2]

    @pl.when(pl.program_id(0) == 0)
    def _init():
        c2t = jnp.transpose(ct_ref[...] * -2.0)
        csq = jnp.sum(c2t * c2t, axis=1, keepdims=True) * 0.25
        h1 = csq.astype(jnp.bfloat16).astype(jnp.float32)
        r1 = csq - h1
        h2 = r1.astype(jnp.bfloat16).astype(jnp.float32)
        h3 = r1 - h2
        c2a_ref[...] = jnp.concatenate([c2t, h1, h2, h3], axis=1)

    c2a = c2a_ref[...]
    ones3 = jnp.ones((3, length), jnp.float32)

    for r in range(_B_ROWS):
        xta = jnp.concatenate([x_ref[r], ones3], axis=0)
        scores = jax.lax.dot_general(
            c2a, xta, (((1,), (0,)), ((), ())),
            preferred_element_type=jnp.float32)
        o_ref[r, :] = jnp.argmin(scores, axis=0).astype(jnp.float32)


def kernel(x, clusters):
    bs, length, dim = x.shape
    k = clusters.shape[0]
    xt = jnp.transpose(x, (0, 2, 1))
    ct = clusters.T
    grid = bs // _B_ROWS
    out = pl.pallas_call(
        _nc_body,
        grid=(grid,),
        in_specs=[
            pl.BlockSpec((_B_ROWS, dim, length), lambda i: (i, 0, 0)),
            pl.BlockSpec((dim, k), lambda i: (0, 0)),
        ],
        out_specs=pl.BlockSpec((_B_ROWS, length), lambda i: (i, 0)),
        out_shape=jax.ShapeDtypeStruct((bs, length), jnp.float32),
        scratch_shapes=[
            pltpu.VMEM((k, dim + 3), jnp.float32),
        ],
    )(xt, ct)
    return out
